# Initial kernel scaffold; baseline (speedup 1.0000x reference)
import jax
import jax.numpy as jnp
from jax.experimental import pallas as pl


def kernel(x, edges, W1, b1, W2, b2):
    raise NotImplementedError("write your pallas kernel here")


# speedup vs baseline: 13.3026x; 13.3026x over previous
import functools

import jax
import jax.numpy as jnp
from jax import lax
from jax.experimental import pallas as pl
from jax.experimental.pallas import tpu as pltpu
from jax.experimental.pallas import tpu_sc as plsc

N = 10000
E = 320000
D_IN = 128
D_HID = 128
D_OUT = 64

NC = 2
NS = 16
NW = NC * NS

N_PAD = 10240
ROWS_PER_TILE = N_PAD // NS
CHUNK = 80
E_PER_TILE = E // NW
N_CHUNKS = E_PER_TILE // CHUNK

_MESH = plsc.VectorSubcoreMesh(core_axis_name="c", subcore_axis_name="s")
_SC_PARAMS = pltpu.CompilerParams(use_tc_tiling_on_sc=False)


def _make_agg_kernel(d):
    @functools.partial(
        pl.kernel,
        mesh=_MESH,
        out_type=jax.ShapeDtypeStruct((NC, N_PAD, d), jnp.float32),
        compiler_params=_SC_PARAMS,
        scratch_types=[
            pltpu.VMEM((CHUNK,), jnp.int32),
            pltpu.VMEM((CHUNK,), jnp.int32),
            pltpu.VMEM((CHUNK, d), jnp.float32),
            pltpu.VMEM_SHARED((N_PAD, d), jnp.float32),
            pltpu.SemaphoreType.DMA,
        ],
    )
    def agg(z_hbm, src_hbm, dst_hbm, zeros_hbm, out_hbm,
            src_v, dst_v, rows_v, acc_sh, sem):
        cid = lax.axis_index("c")
        sid = lax.axis_index("s")
        r0 = sid * ROWS_PER_TILE
        pltpu.sync_copy(zeros_hbm.at[pl.ds(r0, ROWS_PER_TILE)],
                        acc_sh.at[pl.ds(r0, ROWS_PER_TILE)])
        plsc.subcore_barrier()

        wid = sid * NC + cid
        base0 = wid * E_PER_TILE

        def body(c, carry):
            base = base0 + c * CHUNK
            pltpu.sync_copy(src_hbm.at[pl.ds(base, CHUNK)], src_v)
            pltpu.sync_copy(dst_hbm.at[pl.ds(base, CHUNK)], dst_v)
            pltpu.async_copy(z_hbm.at[src_v], rows_v, sem).wait()
            pltpu.sync_copy(rows_v, acc_sh.at[dst_v], add=True)
            return carry

        lax.fori_loop(0, N_CHUNKS, body, 0)
        plsc.subcore_barrier()
        pltpu.sync_copy(acc_sh.at[pl.ds(r0, ROWS_PER_TILE)],
                        out_hbm.at[cid, pl.ds(r0, ROWS_PER_TILE)])

    return agg


def _make_deg_kernel():
    @functools.partial(
        pl.kernel,
        mesh=_MESH,
        out_type=jax.ShapeDtypeStruct((NC, N_PAD, 16), jnp.float32),
        compiler_params=_SC_PARAMS,
        scratch_types=[
            pltpu.VMEM((CHUNK,), jnp.int32),
            pltpu.VMEM((CHUNK, 16), jnp.float32),
            pltpu.VMEM_SHARED((N_PAD, 16), jnp.float32),
        ],
    )
    def deg(dst_hbm, zeros_hbm, out_hbm, dst_v, ones_v, acc_sh):
        cid = lax.axis_index("c")
        sid = lax.axis_index("s")
        r0 = sid * ROWS_PER_TILE
        pltpu.sync_copy(zeros_hbm.at[pl.ds(r0, ROWS_PER_TILE)],
                        acc_sh.at[pl.ds(r0, ROWS_PER_TILE)])
        for j in range(CHUNK):
            ones_v[j, :] = jnp.ones((16,), jnp.float32)
        plsc.subcore_barrier()

        wid = sid * NC + cid
        base0 = wid * E_PER_TILE

        def body(c, carry):
            base = base0 + c * CHUNK
            pltpu.sync_copy(dst_hbm.at[pl.ds(base, CHUNK)], dst_v)
            pltpu.sync_copy(ones_v, acc_sh.at[dst_v], add=True)
            return carry

        lax.fori_loop(0, N_CHUNKS, body, 0)
        plsc.subcore_barrier()
        pltpu.sync_copy(acc_sh.at[pl.ds(r0, ROWS_PER_TILE)],
                        out_hbm.at[cid, pl.ds(r0, ROWS_PER_TILE)])

    return deg


_R = 1000


def _tc1(x, w1, p0, p1):
    def body(x_ref, w_ref, p0_ref, p1_ref, z_ref, dis_ref):
        deg = 1.0 + p0_ref[...] + p1_ref[...]
        disv = lax.rsqrt(deg)
        dis_ref[...] = disv
        mm = jnp.dot(x_ref[...], w_ref[...], preferred_element_type=jnp.float32)
        z_ref[...] = disv[:, 0:1] * mm

    return pl.pallas_call(
        body,
        grid=(N // _R,),
        in_specs=[
            pl.BlockSpec((_R, D_IN), lambda i: (i, 0)),
            pl.BlockSpec((D_IN, D_HID), lambda i: (0, 0)),
            pl.BlockSpec((_R, 16), lambda i: (i, 0)),
            pl.BlockSpec((_R, 16), lambda i: (i, 0)),
        ],
        out_specs=[
            pl.BlockSpec((_R, D_HID), lambda i: (i, 0)),
            pl.BlockSpec((_R, 16), lambda i: (i, 0)),
        ],
        out_shape=[
            jax.ShapeDtypeStruct((N, D_HID), jnp.float32),
            jax.ShapeDtypeStruct((N, 16), jnp.float32),
        ],
    )(x, w1, p0, p1)


def _tc2(agg_a, agg_b, z1, dis, b1, w2):
    def body(a_ref, b_ref, z_ref, dis_ref, b1_ref, w_ref, out_ref):
        pre = dis_ref[:, 0:1] * (a_ref[...] + b_ref[...] + z_ref[...]) + b1_ref[...]
        h = 1.0 / (1.0 + jnp.exp(-pre))
        mm = jnp.dot(h, w_ref[...], preferred_element_type=jnp.float32)
        out_ref[...] = dis_ref[:, 0:1] * mm

    return pl.pallas_call(
        body,
        grid=(N // _R,),
        in_specs=[
            pl.BlockSpec((_R, D_HID), lambda i: (i, 0)),
            pl.BlockSpec((_R, D_HID), lambda i: (i, 0)),
            pl.BlockSpec((_R, D_HID), lambda i: (i, 0)),
            pl.BlockSpec((_R, 16), lambda i: (i, 0)),
            pl.BlockSpec((1, D_HID), lambda i: (0, 0)),
            pl.BlockSpec((D_HID, D_OUT), lambda i: (0, 0)),
        ],
        out_specs=pl.BlockSpec((_R, D_OUT), lambda i: (i, 0)),
        out_shape=jax.ShapeDtypeStruct((N, D_OUT), jnp.float32),
    )(agg_a, agg_b, z1, dis, b1, w2)


def _tc3(agg_a, agg_b, z2, dis, b2):
    def body(a_ref, b_ref, z_ref, dis_ref, b2_ref, out_ref):
        out_ref[...] = (
            dis_ref[:, 0:1] * (a_ref[...] + b_ref[...] + z_ref[...]) + b2_ref[...]
        )

    return pl.pallas_call(
        body,
        grid=(N // _R,),
        in_specs=[
            pl.BlockSpec((_R, D_OUT), lambda i: (i, 0)),
            pl.BlockSpec((_R, D_OUT), lambda i: (i, 0)),
            pl.BlockSpec((_R, D_OUT), lambda i: (i, 0)),
            pl.BlockSpec((_R, 16), lambda i: (i, 0)),
            pl.BlockSpec((1, D_OUT), lambda i: (0, 0)),
        ],
        out_specs=pl.BlockSpec((_R, D_OUT), lambda i: (i, 0)),
        out_shape=jax.ShapeDtypeStruct((N, D_OUT), jnp.float32),
    )(agg_a, agg_b, z2, dis, b2)


_deg_kernel = _make_deg_kernel()
_agg128 = _make_agg_kernel(D_HID)
_agg64 = _make_agg_kernel(D_OUT)


def kernel(x, edges, W1, b1, W2, b2):
    edges = edges.astype(jnp.int32)
    src = edges[0]
    dst = edges[1]
    zeros16 = jnp.zeros((N_PAD, 16), jnp.float32)
    zeros128 = jnp.zeros((N_PAD, D_HID), jnp.float32)
    zeros64 = jnp.zeros((N_PAD, D_OUT), jnp.float32)
    b1f = b1.reshape(1, D_HID)
    b2f = b2.reshape(1, D_OUT)

    degp = _deg_kernel(dst, zeros16)
    p0 = degp[0, :N]
    p1 = degp[1, :N]

    z1, dis = _tc1(x, W1, p0, p1)
    agg1 = _agg128(z1, src, dst, zeros128)
    z2 = _tc2(agg1[0, :N], agg1[1, :N], z1, dis, b1f, W2)
    agg2 = _agg64(z2, src, dst, zeros64)
    out = _tc3(agg2[0, :N], agg2[1, :N], z2, dis, b2f)
    return out


# speedup vs baseline: 30.7378x; 2.3107x over previous
import functools

import jax
import jax.numpy as jnp
from jax import lax
from jax.experimental import pallas as pl
from jax.experimental.pallas import tpu as pltpu
from jax.experimental.pallas import tpu_sc as plsc

N = 10000
E = 320000
D_IN = 128
D_HID = 128
D_OUT = 64

NC = 2
NS = 16
NW = NC * NS

N_PAD = 10000
ROWS_PER_TILE = N_PAD // NS
CHUNK = 100
E_PER_TILE = E // NW
N_CHUNKS = E_PER_TILE // CHUNK

_MESH = plsc.VectorSubcoreMesh(core_axis_name="c", subcore_axis_name="s")
_SC_PARAMS = pltpu.CompilerParams(use_tc_tiling_on_sc=False)


NBUF = 2


def _make_agg_kernel(d):
    @functools.partial(
        pl.kernel,
        mesh=_MESH,
        out_type=jax.ShapeDtypeStruct((NC, N_PAD, d), jnp.float32),
        compiler_params=_SC_PARAMS,
        scratch_types=[
            pltpu.VMEM((N_CHUNKS, CHUNK), jnp.int32) @ _MESH,
            pltpu.VMEM((N_CHUNKS, CHUNK), jnp.int32) @ _MESH,
            [pltpu.VMEM((CHUNK, d), jnp.float32) @ _MESH] * NBUF,
            pltpu.VMEM_SHARED((N_PAD, d), jnp.float32),
            [pltpu.SemaphoreType.DMA @ _MESH] * NBUF,
        ],
    )
    def agg(z_hbm, src_hbm, dst_hbm, zeros_hbm, out_hbm,
            src_all, dst_all, rows, acc_sh, sems):
        cid = lax.axis_index("c")
        sid = lax.axis_index("s")
        r0 = sid * ROWS_PER_TILE
        wid = sid * NC + cid
        c0 = wid * N_CHUNKS
        pltpu.sync_copy(src_hbm.at[pl.ds(c0, N_CHUNKS)], src_all)
        pltpu.sync_copy(dst_hbm.at[pl.ds(c0, N_CHUNKS)], dst_all)
        pltpu.sync_copy(zeros_hbm.at[pl.ds(r0, ROWS_PER_TILE)],
                        acc_sh.at[pl.ds(r0, ROWS_PER_TILE)])
        plsc.subcore_barrier()

        for b in range(NBUF):
            pltpu.async_copy(z_hbm.at[src_all.at[b]], rows[b], sems[b])

        def body(c_base, carry):
            for b in range(NBUF):
                c = c_base + b
                pltpu.make_async_copy(
                    z_hbm.at[src_all.at[c]], rows[b], sems[b]).wait()
                pltpu.sync_copy(rows[b], acc_sh.at[dst_all.at[c]], add=True)
                nxt = c + NBUF

                @pl.when(nxt < N_CHUNKS)
                def _():
                    pltpu.async_copy(z_hbm.at[src_all.at[nxt]], rows[b], sems[b])

            return carry

        lax.fori_loop(0, N_CHUNKS // NBUF, lambda i, car: body(i * NBUF, car), 0)
        plsc.subcore_barrier()
        pltpu.sync_copy(acc_sh.at[pl.ds(r0, ROWS_PER_TILE)],
                        out_hbm.at[cid, pl.ds(r0, ROWS_PER_TILE)])

    return agg


def _make_deg_kernel():
    @functools.partial(
        pl.kernel,
        mesh=_MESH,
        out_type=jax.ShapeDtypeStruct((NC, N_PAD, 16), jnp.float32),
        compiler_params=_SC_PARAMS,
        scratch_types=[
            pltpu.VMEM((N_CHUNKS, CHUNK), jnp.int32),
            pltpu.VMEM((CHUNK, 16), jnp.float32),
            pltpu.VMEM_SHARED((N_PAD, 16), jnp.float32),
            pltpu.SemaphoreType.DMA,
        ],
    )
    def deg(dst_hbm, zeros_hbm, out_hbm, dst_all, ones_v, acc_sh, sem):
        cid = lax.axis_index("c")
        sid = lax.axis_index("s")
        r0 = sid * ROWS_PER_TILE
        wid = sid * NC + cid
        c0 = wid * N_CHUNKS
        pltpu.sync_copy(dst_hbm.at[pl.ds(c0, N_CHUNKS)], dst_all)
        pltpu.sync_copy(zeros_hbm.at[pl.ds(r0, ROWS_PER_TILE)],
                        acc_sh.at[pl.ds(r0, ROWS_PER_TILE)])
        for j in range(CHUNK):
            ones_v[j, :] = jnp.ones((16,), jnp.float32)
        plsc.subcore_barrier()

        def body(c_base, carry):
            for b in range(NBUF):
                pltpu.async_copy(
                    ones_v, acc_sh.at[dst_all.at[c_base + b]], sem, add=True)
            for b in range(NBUF):
                pltpu.make_async_copy(
                    ones_v, acc_sh.at[dst_all.at[c_base + b]], sem).wait()
            return carry

        lax.fori_loop(0, N_CHUNKS // NBUF, lambda i, car: body(i * NBUF, car), 0)
        plsc.subcore_barrier()
        pltpu.sync_copy(acc_sh.at[pl.ds(r0, ROWS_PER_TILE)],
                        out_hbm.at[cid, pl.ds(r0, ROWS_PER_TILE)])

    return deg


_R = 1000


def _tc1(x, w1, p0, p1):
    def body(x_ref, w_ref, p0_ref, p1_ref, z_ref, dis_ref):
        deg = 1.0 + p0_ref[...] + p1_ref[...]
        disv = lax.rsqrt(deg)
        dis_ref[...] = disv
        mm = jnp.dot(x_ref[...], w_ref[...], preferred_element_type=jnp.float32)
        z_ref[...] = disv[:, 0:1] * mm

    return pl.pallas_call(
        body,
        grid=(N // _R,),
        in_specs=[
            pl.BlockSpec((_R, D_IN), lambda i: (i, 0)),
            pl.BlockSpec((D_IN, D_HID), lambda i: (0, 0)),
            pl.BlockSpec((_R, 16), lambda i: (i, 0)),
            pl.BlockSpec((_R, 16), lambda i: (i, 0)),
        ],
        out_specs=[
            pl.BlockSpec((_R, D_HID), lambda i: (i, 0)),
            pl.BlockSpec((_R, 16), lambda i: (i, 0)),
        ],
        out_shape=[
            jax.ShapeDtypeStruct((N, D_HID), jnp.float32),
            jax.ShapeDtypeStruct((N, 16), jnp.float32),
        ],
    )(x, w1, p0, p1)


def _tc2(agg_a, agg_b, z1, dis, b1, w2):
    def body(a_ref, b_ref, z_ref, dis_ref, b1_ref, w_ref, out_ref):
        pre = dis_ref[:, 0:1] * (a_ref[...] + b_ref[...] + z_ref[...]) + b1_ref[...]
        h = 1.0 / (1.0 + jnp.exp(-pre))
        mm = jnp.dot(h, w_ref[...], preferred_element_type=jnp.float32)
        out_ref[...] = dis_ref[:, 0:1] * mm

    return pl.pallas_call(
        body,
        grid=(N // _R,),
        in_specs=[
            pl.BlockSpec((_R, D_HID), lambda i: (i, 0)),
            pl.BlockSpec((_R, D_HID), lambda i: (i, 0)),
            pl.BlockSpec((_R, D_HID), lambda i: (i, 0)),
            pl.BlockSpec((_R, 16), lambda i: (i, 0)),
            pl.BlockSpec((1, D_HID), lambda i: (0, 0)),
            pl.BlockSpec((D_HID, D_OUT), lambda i: (0, 0)),
        ],
        out_specs=pl.BlockSpec((_R, D_OUT), lambda i: (i, 0)),
        out_shape=jax.ShapeDtypeStruct((N, D_OUT), jnp.float32),
    )(agg_a, agg_b, z1, dis, b1, w2)


def _tc3(agg_a, agg_b, z2, dis, b2):
    def body(a_ref, b_ref, z_ref, dis_ref, b2_ref, out_ref):
        out_ref[...] = (
            dis_ref[:, 0:1] * (a_ref[...] + b_ref[...] + z_ref[...]) + b2_ref[...]
        )

    return pl.pallas_call(
        body,
        grid=(N // _R,),
        in_specs=[
            pl.BlockSpec((_R, D_OUT), lambda i: (i, 0)),
            pl.BlockSpec((_R, D_OUT), lambda i: (i, 0)),
            pl.BlockSpec((_R, D_OUT), lambda i: (i, 0)),
            pl.BlockSpec((_R, 16), lambda i: (i, 0)),
            pl.BlockSpec((1, D_OUT), lambda i: (0, 0)),
        ],
        out_specs=pl.BlockSpec((_R, D_OUT), lambda i: (i, 0)),
        out_shape=jax.ShapeDtypeStruct((N, D_OUT), jnp.float32),
    )(agg_a, agg_b, z2, dis, b2)


_deg_kernel = _make_deg_kernel()
_agg128 = _make_agg_kernel(D_HID)
_agg64 = _make_agg_kernel(D_OUT)


def kernel(x, edges, W1, b1, W2, b2):
    edges = edges.astype(jnp.int32)
    src = edges[0].reshape(E // CHUNK, CHUNK)
    dst = edges[1].reshape(E // CHUNK, CHUNK)
    zeros16 = jnp.zeros((N_PAD, 16), jnp.float32)
    zeros128 = jnp.zeros((N_PAD, D_HID), jnp.float32)
    zeros64 = jnp.zeros((N_PAD, D_OUT), jnp.float32)
    b1f = b1.reshape(1, D_HID)
    b2f = b2.reshape(1, D_OUT)

    degp = _deg_kernel(dst, zeros16)
    p0 = degp[0, :N]
    p1 = degp[1, :N]

    z1, dis = _tc1(x, W1, p0, p1)
    agg1 = _agg128(z1, src, dst, zeros128)
    z2 = _tc2(agg1[0, :N], agg1[1, :N], z1, dis, b1f, W2)
    agg2 = _agg64(z2, src, dst, zeros64)
    out = _tc3(agg2[0, :N], agg2[1, :N], z2, dis, b2f)
    return out
